# Initial kernel scaffold; baseline (speedup 1.0000x reference)
import jax
import jax.numpy as jnp
from jax.experimental import pallas as pl


def kernel(tokens, edge_index, edge_type, mask, embed, enc_W, enc_b, skip_W, film_skip_W, lins_W, films_W, films_b, gru_W, gru_b, W3, W3_b, w3, w3_b, mlp_W, mlp_b):
    raise NotImplementedError("write your pallas kernel here")


# speedup vs baseline: 6.7276x; 6.7276x over previous
import functools

import jax
import jax.numpy as jnp
from jax.experimental import pallas as pl

_R = 4


def _encode_body(x_ref, w_ref, b_ref, o_ref):
    o_ref[...] = jax.nn.relu(
        jnp.dot(x_ref[...], w_ref[...], preferred_element_type=jnp.float32)
        + b_ref[...]
    )


def _nodepre_body(x_ref, skip_ref, fskip_ref, lins_ref, films_ref, fb_ref,
                  a0_ref, xj_ref, fbo_ref):
    x = x_ref[...]
    d = x.shape[1]
    fs = jnp.dot(x, fskip_ref[...], preferred_element_type=jnp.float32)
    beta_s, gamma_s = fs[:, :d], fs[:, d:]
    xs = jnp.dot(x, skip_ref[...], preferred_element_type=jnp.float32)
    a0_ref[...] = jax.nn.relu(gamma_s * xs + beta_s)
    xj_ref[...] = jnp.dot(x, lins_ref[...], preferred_element_type=jnp.float32)
    fbo_ref[...] = (
        jnp.dot(x, films_ref[...], preferred_element_type=jnp.float32)
        + fb_ref[...]
    )


def _msg_body(xj_ref, fb_ref, o_ref):
    xj = xj_ref[...]
    d = xj.shape[1]
    fb = fb_ref[...]
    o_ref[...] = jax.nn.relu(fb[:, d:] * xj + fb[:, :d])


def _attn_body(x_ref, m_ref, o_ref, *, nsen):
    for i in range(nsen):
        s = x_ref[i] * m_ref[i]
        ell = s.shape[0]
        scores = jax.lax.dot_general(
            s, s, (((1,), (1,)), ((), ())),
            preferred_element_type=jnp.float32,
        ) * (1.0 / (s.shape[1] ** 0.5))
        mx = jnp.max(scores, axis=1, keepdims=True)
        e = jnp.exp(scores - mx)
        p = e / jnp.sum(e, axis=1, keepdims=True)
        w = jnp.sum(p, axis=0, keepdims=True)
        o_ref[i:i + 1, :] = jnp.dot(
            w, s, preferred_element_type=jnp.float32) * (1.0 / ell)


def _gru_body(a0_ref, sm_ref, rcp_ref, x_ref, sen_ref,
              wa_ref, wx_ref, wh1_ref, ws_ref, b_ref, o_ref, *, nsen):
    x = x_ref[...]
    d = x.shape[1]
    a = a0_ref[...]
    sm = sm_ref[...]
    rcp = rcp_ref[...]
    for r in range(_R):
        a = a + sm[:, r * d:(r + 1) * d] * rcp[:, r:r + 1]
    ta = jnp.dot(a, wa_ref[...], preferred_element_type=jnp.float32)
    tx = jnp.dot(x, wx_ref[...], preferred_element_type=jnp.float32)
    ts = jnp.dot(sen_ref[...], ws_ref[...], preferred_element_type=jnp.float32)
    b = b_ref[...]
    ell = x.shape[0] // nsen
    for i in range(nsen):
        sl = slice(i * ell, (i + 1) * ell)
        tsi = ts[i:i + 1, :]
        z = jax.nn.sigmoid(
            ta[sl, :d] + tx[sl, :d] + tsi[:, :d] + b[:, :d])
        rg = jax.nn.sigmoid(
            ta[sl, d:2 * d] + tx[sl, d:2 * d] + tsi[:, d:2 * d]
            + b[:, d:2 * d])
        h = jax.nn.relu(
            ta[sl, 2 * d:] + jnp.dot(
                x[sl, :] * rg, wh1_ref[...],
                preferred_element_type=jnp.float32)
            + tsi[:, 2 * d:] + b[:, 2 * d:])
        o_ref[sl, :] = h * z + x[sl, :] * (1.0 - z)


def _readout_body(x_ref, m_ref, w3_ref, w3b_ref, v_ref, vb_ref,
                  mw_ref, mb_ref, o_ref, *, nsen):
    for i in range(nsen):
        h = x_ref[i] * m_ref[i]
        t = jnp.tanh(
            jnp.dot(h, w3_ref[...], preferred_element_type=jnp.float32)
            + w3b_ref[...])
        c = jnp.dot(t, v_ref[...], preferred_element_type=jnp.float32) \
            + vb_ref[...]
        mx = jnp.max(c, axis=0, keepdims=True)
        e = jnp.exp(c - mx)
        att = e / jnp.sum(e, axis=0, keepdims=True)
        aout = jnp.sum(att * h, axis=0, keepdims=True)
        o_ref[i:i + 1, :] = jnp.dot(
            aout, mw_ref[...], preferred_element_type=jnp.float32) \
            + mb_ref[...]


def _full(shape):
    return pl.BlockSpec(shape, lambda i: (0,) * len(shape))


def kernel(tokens, edge_index, edge_type, mask, embed, enc_W, enc_b,
           skip_W, film_skip_W, lins_W, films_W, films_b, gru_W, gru_b,
           W3, W3_b, w3, w3_b, mlp_W, mlp_b):
    n = tokens.shape[0]
    b_, l_, _ = mask.shape
    d = embed.shape[1]
    steps = skip_W.shape[0]
    c_out = mlp_W.shape[1]
    f32 = jnp.float32

    bn = min(2048, n)
    be = min(8192, edge_type.shape[0])
    sb = 8
    rows = sb * l_

    lins_cat = [jnp.transpose(lins_W[s], (1, 0, 2)).reshape(d, _R * d)
                for s in range(steps)]
    films_cat = [jnp.transpose(films_W[s], (1, 0, 2)).reshape(d, _R * 2 * d)
                 for s in range(steps)]
    films_bcat = [films_b[s].reshape(1, _R * 2 * d) for s in range(steps)]
    wa = [jnp.concatenate([gru_W[s, 0], gru_W[s, 3], gru_W[s, 6]], axis=1)
          for s in range(steps)]
    wx = [jnp.concatenate([gru_W[s, 1], gru_W[s, 4]], axis=1)
          for s in range(steps)]
    ws = [jnp.concatenate([gru_W[s, 2], gru_W[s, 5], gru_W[s, 8]], axis=1)
          for s in range(steps)]
    wh1 = [gru_W[s, 7] for s in range(steps)]
    b3 = [jnp.concatenate(
        [gru_b[s, 0] + gru_b[s, 1] + gru_b[s, 2],
         gru_b[s, 3] + gru_b[s, 4] + gru_b[s, 5],
         gru_b[s, 6] + gru_b[s, 7] + gru_b[s, 8]]).reshape(1, 3 * d)
        for s in range(steps)]

    x = jnp.take(embed, tokens, axis=0)
    x = pl.pallas_call(
        _encode_body,
        grid=(n // bn,),
        in_specs=[pl.BlockSpec((bn, d), lambda i: (i, 0)),
                  _full((d, d)), _full((1, d))],
        out_specs=pl.BlockSpec((bn, d), lambda i: (i, 0)),
        out_shape=jax.ShapeDtypeStruct((n, d), f32),
    )(x, enc_W, enc_b.reshape(1, d))

    src, dst = edge_index[0], edge_index[1]
    et = edge_type.astype(src.dtype)
    g_src = src * _R + et
    g_dst = dst * _R + et
    cnt = jnp.zeros((n * _R,), f32).at[g_dst].add(1.0)
    rcp = (1.0 / jnp.maximum(cnt, 1.0)).reshape(n, _R)

    for s in range(steps):
        senten = pl.pallas_call(
            functools.partial(_attn_body, nsen=sb),
            grid=(b_ // sb,),
            in_specs=[pl.BlockSpec((sb, l_, d), lambda i: (i, 0, 0)),
                      pl.BlockSpec((sb, l_, 1), lambda i: (i, 0, 0))],
            out_specs=pl.BlockSpec((sb, d), lambda i: (i, 0)),
            out_shape=jax.ShapeDtypeStruct((b_, d), f32),
        )(x.reshape(b_, l_, d), mask)

        a0, xj_all, fb_all = pl.pallas_call(
            _nodepre_body,
            grid=(n // bn,),
            in_specs=[pl.BlockSpec((bn, d), lambda i: (i, 0)),
                      _full((d, d)), _full((d, 2 * d)),
                      _full((d, _R * d)), _full((d, _R * 2 * d)),
                      _full((1, _R * 2 * d))],
            out_specs=(pl.BlockSpec((bn, d), lambda i: (i, 0)),
                       pl.BlockSpec((bn, _R * d), lambda i: (i, 0)),
                       pl.BlockSpec((bn, _R * 2 * d), lambda i: (i, 0))),
            out_shape=(jax.ShapeDtypeStruct((n, d), f32),
                       jax.ShapeDtypeStruct((n, _R * d), f32),
                       jax.ShapeDtypeStruct((n, _R * 2 * d), f32)),
        )(x, skip_W[s], film_skip_W[s], lins_cat[s], films_cat[s],
          films_bcat[s])

        xj_e = xj_all.reshape(n * _R, d)[g_src]
        fb_e = fb_all.reshape(n * _R, 2 * d)[g_dst]
        msg = pl.pallas_call(
            _msg_body,
            grid=(xj_e.shape[0] // be,),
            in_specs=[pl.BlockSpec((be, d), lambda i: (i, 0)),
                      pl.BlockSpec((be, 2 * d), lambda i: (i, 0))],
            out_specs=pl.BlockSpec((be, d), lambda i: (i, 0)),
            out_shape=jax.ShapeDtypeStruct((xj_e.shape[0], d), f32),
        )(xj_e, fb_e)
        summed = jnp.zeros((n * _R, d), f32).at[g_dst].add(msg)
        summed = summed.reshape(n, _R * d)

        x = pl.pallas_call(
            functools.partial(_gru_body, nsen=sb),
            grid=(n // rows,),
            in_specs=[pl.BlockSpec((rows, d), lambda i: (i, 0)),
                      pl.BlockSpec((rows, _R * d), lambda i: (i, 0)),
                      pl.BlockSpec((rows, _R), lambda i: (i, 0)),
                      pl.BlockSpec((rows, d), lambda i: (i, 0)),
                      pl.BlockSpec((sb, d), lambda i: (i, 0)),
                      _full((d, 3 * d)), _full((d, 2 * d)),
                      _full((d, d)), _full((d, 3 * d)),
                      _full((1, 3 * d))],
            out_specs=pl.BlockSpec((rows, d), lambda i: (i, 0)),
            out_shape=jax.ShapeDtypeStruct((n, d), f32),
        )(a0, summed, rcp, x, senten, wa[s], wx[s], wh1[s], ws[s], b3[s])

    out = pl.pallas_call(
        functools.partial(_readout_body, nsen=sb),
        grid=(b_ // sb,),
        in_specs=[pl.BlockSpec((sb, l_, d), lambda i: (i, 0, 0)),
                  pl.BlockSpec((sb, l_, 1), lambda i: (i, 0, 0)),
                  _full((d, d)), _full((1, d)),
                  _full((d, 1)), _full((1, 1)),
                  _full((d, c_out)), _full((1, c_out))],
        out_specs=pl.BlockSpec((sb, c_out), lambda i: (i, 0)),
        out_shape=jax.ShapeDtypeStruct((b_, c_out), f32),
    )(x.reshape(b_, l_, d), mask, W3, W3_b.reshape(1, d),
      w3, w3_b.reshape(1, 1), mlp_W, mlp_b.reshape(1, c_out))
    return out
